# baseline (speedup 1.0000x reference)
import functools

import jax
import jax.numpy as jnp
from jax import lax
from jax.experimental import pallas as pl
from jax.experimental.pallas import tpu as pltpu
from jax.experimental.pallas import tpu_sc as plsc

_K = 8
_N = 4096
_NW = 32
_RS = 1664
_WPB = 8
_RPW = _RS // _WPB
_NCH = _N // 16
_ROWS = 128
_INF = float("inf")


def _bf16_round(x):
    i = lax.bitcast_convert_type(x, jnp.int32)
    odd = lax.shift_right_logical(i, 16) & jnp.int32(1)
    i = i + jnp.int32(0x7FFF) + odd
    i = i & jnp.int32(-65536)
    return lax.bitcast_convert_type(i, jnp.float32)


def _fsqrt(x):
    i = lax.bitcast_convert_type(x, jnp.int32)
    i = jnp.int32(0x1FBD1DF5) + lax.shift_right_arithmetic(i, 1)
    y = lax.bitcast_convert_type(i, jnp.float32)
    for _ in range(3):
        y = jnp.float32(0.5) * (y + x / y)
    return y


def _sort_asc(v):
    return plsc.sort_key_val(v, v)[0]


def _sort_desc(v):
    return plsc.sort_key_val(v, v, descending=True)[0]


def _merge_sorted(regs):
    b = _sort_asc(regs[0])
    for m in regs[1:]:
        b = _sort_asc(jnp.minimum(b, _sort_desc(m)))
    return b


def _bubble(ms, v):
    out = []
    for m in ms:
        lo = jnp.minimum(m, v)
        v = jnp.maximum(m, v)
        out.append(lo)
    return tuple(out)


def _sc_body(xt_hbm, out_hbm, xs, ys, zs, ss, xr, yr, zr, accs):
    wid = lax.axis_index("s") * 2 + lax.axis_index("c")
    batch = wid // _WPB
    rowbase = (wid % _WPB) * _RPW

    base = batch * 3 * _N
    pltpu.sync_copy(xt_hbm.at[pl.ds(base, _N)], xs)
    pltpu.sync_copy(xt_hbm.at[pl.ds(base + _N, _N)], ys)
    pltpu.sync_copy(xt_hbm.at[pl.ds(base + 2 * _N, _N)], zs)

    lanes = lax.iota(jnp.int32, 16)
    inf16 = jnp.full((16,), _INF, jnp.float32)

    def _norms(j, carry):
        sl = pl.ds(j * 16, 16)
        xv, yv, zv = xs[sl], ys[sl], zs[sl]
        ss[sl] = xv * xv + yv * yv + zv * zv
        xr[sl] = _bf16_round(xv)
        yr[sl] = _bf16_round(yv)
        zr[sl] = _bf16_round(zv)
        return carry

    lax.fori_loop(0, _NCH, _norms, 0)

    def _row(r, acc):
        ridx = jnp.full((16,), rowbase + r, jnp.int32)
        qx = plsc.load_gather(xs, [ridx])
        qy = plsc.load_gather(ys, [ridx])
        qz = plsc.load_gather(zs, [ridx])
        ax = jnp.float32(-2.0) * plsc.load_gather(xr, [ridx])
        ay = jnp.float32(-2.0) * plsc.load_gather(yr, [ridx])
        az = jnp.float32(-2.0) * plsc.load_gather(zr, [ridx])
        sq_q = qx * qx + qy * qy + qz * qz

        def _chunk4(j, ms):
            sl = pl.ds(j * 16, 16)
            v = ss[sl] + ax * xr[sl] + ay * yr[sl] + az * zr[sl]
            return _bubble(ms, v)

        ms4 = lax.fori_loop(0, _NCH, _chunk4, (inf16,) * 4, unroll=8)
        b = _merge_sorted(list(ms4))
        v8 = b[7]
        suspect = plsc.all_reduce_population_count(ms4[3] <= v8)[0]

        def _exact(_):
            def _chunk8(j, ms):
                sl = pl.ds(j * 16, 16)
                v = ss[sl] + ax * xr[sl] + ay * yr[sl] + az * zr[sl]
                return _bubble(ms, v)

            ms8 = lax.fori_loop(0, _NCH, _chunk8, (inf16,) * 8, unroll=4)
            return _merge_sorted(list(ms8))

        vals = lax.cond(suspect > 0, _exact, lambda _: b, 0)

        d2 = jnp.maximum(vals + sq_q, jnp.float32(0.0))
        good = jnp.logical_and(d2 > 0, lanes < _K)
        root = jnp.where(good, _fsqrt(jnp.where(good, d2, 1.0)), 0.0)
        return acc + root

    acc = lax.fori_loop(0, _RPW, _row, jnp.zeros((16,), jnp.float32))
    accs[...] = acc
    pltpu.sync_copy(accs, out_hbm.at[wid])


def _tc_extract(regs, inf):
    rows = regs[0].shape[0]
    acc = jnp.zeros((rows, 1), jnp.float32)
    needed = jnp.full((rows, 1), float(_K), jnp.float32)
    m = None
    for _ in range(_K):
        mm = regs[0]
        for reg in regs[1:]:
            mm = jnp.minimum(mm, reg)
        m = jnp.min(mm, axis=1, keepdims=True)
        eqs = [reg == m for reg in regs]
        cnt = jnp.zeros((rows, 1), jnp.float32)
        for eq in eqs:
            cnt = cnt + jnp.sum(eq.astype(jnp.float32), axis=1, keepdims=True)
        take = jnp.minimum(cnt, needed)
        root = jnp.where(m > 0, jnp.sqrt(jnp.where(m > 0, m, 1.0)), 0.0)
        acc = acc + jnp.where(take > 0, take * root, 0.0)
        needed = needed - take
        regs = [jnp.where(eq, inf, reg) for reg, eq in zip(regs, eqs)]
    return acc, m


def _tc_bubble(d2, depth, inf):
    nch = d2.shape[1] // 128
    regs = [jnp.full((d2.shape[0], 128), inf, jnp.float32)
            for _ in range(depth)]
    for c in range(nch):
        v = d2[:, c * 128:(c + 1) * 128]
        out = []
        for m in regs:
            lo = jnp.minimum(m, v)
            v = jnp.maximum(m, v)
            out.append(lo)
        regs = out
    return regs


def _tc_body(q_ref, yt_ref, out_ref):
    bi = pl.program_id(0)
    ji = pl.program_id(1)
    q = q_ref[0]
    yt = yt_ref[0]
    sq_q = jnp.sum(q * q, axis=1, keepdims=True)
    sq_y = jnp.sum(yt * yt, axis=0, keepdims=True)
    cross = jax.lax.dot_general(
        q, yt, (((1,), (0,)), ((), ())),
        preferred_element_type=jnp.float32)
    d2 = jnp.maximum(sq_q + sq_y - 2.0 * cross, 0.0)

    inf = jnp.float32(jnp.inf)
    regs4 = _tc_bubble(d2, 4, inf)
    acc, v8 = _tc_extract(list(regs4), inf)
    suspect = jnp.max(jnp.where(regs4[3] <= v8, 1.0, 0.0))

    @pl.when(jnp.logical_and(bi == 0, ji == 0))
    def _():
        out_ref[...] = jnp.zeros_like(out_ref)

    out_ref[...] += acc

    @pl.when(suspect > 0)
    def _():
        acc8, _unused = _tc_extract(_tc_bubble(d2, _K, inf), inf)
        out_ref[...] += acc8 - acc


def kernel(generated):
    generated = generated.astype(jnp.float32)
    b, c, h, w = generated.shape
    n = h * w
    xt = generated.reshape(b * c * n)

    yt_tc = generated.reshape(b, c, n)
    q_tc = jnp.transpose(yt_tc, (0, 2, 1))
    j0 = _RS // _ROWS
    tc_part = pl.pallas_call(
        _tc_body,
        grid=(b, (n - _RS) // _ROWS),
        in_specs=[
            pl.BlockSpec((1, _ROWS, c), lambda i, j: (i, j + j0, 0)),
            pl.BlockSpec((1, c, n), lambda i, j: (i, 0, 0)),
        ],
        out_specs=pl.BlockSpec((_ROWS, 1), lambda i, j: (0, 0)),
        out_shape=jax.ShapeDtypeStruct((_ROWS, 1), jnp.float32),
    )(q_tc, yt_tc)

    mesh = plsc.VectorSubcoreMesh(core_axis_name="c", subcore_axis_name="s")
    run = pl.kernel(
        _sc_body,
        mesh=mesh,
        compiler_params=pltpu.CompilerParams(needs_layout_passes=False),
        out_type=jax.ShapeDtypeStruct((_NW, 16), jnp.float32),
        scratch_types=[
            pltpu.VMEM((n,), jnp.float32),
            pltpu.VMEM((n,), jnp.float32),
            pltpu.VMEM((n,), jnp.float32),
            pltpu.VMEM((n,), jnp.float32),
            pltpu.VMEM((n,), jnp.float32),
            pltpu.VMEM((n,), jnp.float32),
            pltpu.VMEM((n,), jnp.float32),
            pltpu.VMEM((16,), jnp.float32),
        ],
    )
    partial = run(xt)
    total = jnp.sum(partial) + jnp.sum(tc_part)
    return -total / jnp.float32(b * n * _K)


# speedup vs baseline: 1.0854x; 1.0854x over previous
import functools

import jax
import jax.numpy as jnp
from jax import lax
from jax.experimental import pallas as pl
from jax.experimental.pallas import tpu as pltpu
from jax.experimental.pallas import tpu_sc as plsc

_K = 8
_N = 4096
_NW = 32
_RS = 1536
_WPB = 8
_RPW = _RS // _WPB
_NCH = _N // 16
_ROWS = 512
_INF = float("inf")


def _bf16_round(x):
    i = lax.bitcast_convert_type(x, jnp.int32)
    odd = lax.shift_right_logical(i, 16) & jnp.int32(1)
    i = i + jnp.int32(0x7FFF) + odd
    i = i & jnp.int32(-65536)
    return lax.bitcast_convert_type(i, jnp.float32)


def _fsqrt(x):
    i = lax.bitcast_convert_type(x, jnp.int32)
    i = jnp.int32(0x1FBD1DF5) + lax.shift_right_arithmetic(i, 1)
    y = lax.bitcast_convert_type(i, jnp.float32)
    for _ in range(3):
        y = jnp.float32(0.5) * (y + x / y)
    return y


def _sort_asc(v):
    return plsc.sort_key_val(v, v)[0]


def _sort_desc(v):
    return plsc.sort_key_val(v, v, descending=True)[0]


def _merge_sorted(regs):
    b = _sort_asc(regs[0])
    for m in regs[1:]:
        b = _sort_asc(jnp.minimum(b, _sort_desc(m)))
    return b


def _bubble(ms, v):
    out = []
    for m in ms:
        lo = jnp.minimum(m, v)
        v = jnp.maximum(m, v)
        out.append(lo)
    return tuple(out)


def _sc_body(xt_hbm, out_hbm, xs, ys, zs, ss, xr, yr, zr, accs):
    wid = lax.axis_index("s") * 2 + lax.axis_index("c")
    batch = wid // _WPB
    rowbase = (wid % _WPB) * _RPW

    base = batch * 3 * _N
    pltpu.sync_copy(xt_hbm.at[pl.ds(base, _N)], xs)
    pltpu.sync_copy(xt_hbm.at[pl.ds(base + _N, _N)], ys)
    pltpu.sync_copy(xt_hbm.at[pl.ds(base + 2 * _N, _N)], zs)

    lanes = lax.iota(jnp.int32, 16)
    inf16 = jnp.full((16,), _INF, jnp.float32)

    def _norms(j, carry):
        sl = pl.ds(j * 16, 16)
        xv, yv, zv = xs[sl], ys[sl], zs[sl]
        ss[sl] = xv * xv + yv * yv + zv * zv
        xr[sl] = _bf16_round(xv)
        yr[sl] = _bf16_round(yv)
        zr[sl] = _bf16_round(zv)
        return carry

    lax.fori_loop(0, _NCH, _norms, 0)

    def _row(r, acc):
        ridx = jnp.full((16,), rowbase + r, jnp.int32)
        qx = plsc.load_gather(xs, [ridx])
        qy = plsc.load_gather(ys, [ridx])
        qz = plsc.load_gather(zs, [ridx])
        ax = jnp.float32(-2.0) * plsc.load_gather(xr, [ridx])
        ay = jnp.float32(-2.0) * plsc.load_gather(yr, [ridx])
        az = jnp.float32(-2.0) * plsc.load_gather(zr, [ridx])
        sq_q = qx * qx + qy * qy + qz * qz

        def _chunk4(j, ms):
            sl = pl.ds(j * 16, 16)
            v = ss[sl] + ax * xr[sl] + ay * yr[sl] + az * zr[sl]
            return _bubble(ms, v)

        ms4 = lax.fori_loop(0, _NCH, _chunk4, (inf16,) * 4, unroll=8)
        b = _merge_sorted(list(ms4))
        v8 = b[7]
        suspect = plsc.all_reduce_population_count(ms4[3] <= v8)[0]

        def _exact(_):
            def _chunk8(j, ms):
                sl = pl.ds(j * 16, 16)
                v = ss[sl] + ax * xr[sl] + ay * yr[sl] + az * zr[sl]
                return _bubble(ms, v)

            ms8 = lax.fori_loop(0, _NCH, _chunk8, (inf16,) * 8, unroll=4)
            return _merge_sorted(list(ms8))

        vals = lax.cond(suspect > 0, _exact, lambda _: b, 0)

        d2 = jnp.maximum(vals + sq_q, jnp.float32(0.0))
        good = jnp.logical_and(d2 > 0, lanes < _K)
        root = jnp.where(good, _fsqrt(jnp.where(good, d2, 1.0)), 0.0)
        return acc + root

    acc = lax.fori_loop(0, _RPW, _row, jnp.zeros((16,), jnp.float32))
    accs[...] = acc
    pltpu.sync_copy(accs, out_hbm.at[wid])


def _tc_extract(regs, inf):
    rows = regs[0].shape[0]
    acc = jnp.zeros((rows, 1), jnp.float32)
    needed = jnp.full((rows, 1), float(_K), jnp.float32)
    m = None
    for _ in range(_K):
        mm = regs[0]
        for reg in regs[1:]:
            mm = jnp.minimum(mm, reg)
        m = jnp.min(mm, axis=1, keepdims=True)
        eqs = [reg == m for reg in regs]
        cnt = jnp.zeros((rows, 1), jnp.float32)
        for eq in eqs:
            cnt = cnt + jnp.sum(eq.astype(jnp.float32), axis=1, keepdims=True)
        take = jnp.minimum(cnt, needed)
        root = jnp.where(m > 0, jnp.sqrt(jnp.where(m > 0, m, 1.0)), 0.0)
        acc = acc + jnp.where(take > 0, take * root, 0.0)
        needed = needed - take
        regs = [jnp.where(eq, inf, reg) for reg, eq in zip(regs, eqs)]
    return acc, m


def _tc_bubble(d2, depth, inf):
    nch = d2.shape[1] // 128
    regs = [jnp.full((d2.shape[0], 128), inf, jnp.float32)
            for _ in range(depth)]
    for c in range(nch):
        v = d2[:, c * 128:(c + 1) * 128]
        out = []
        for m in regs:
            lo = jnp.minimum(m, v)
            v = jnp.maximum(m, v)
            out.append(lo)
        regs = out
    return regs


def _tc_body(q_ref, yt_ref, out_ref):
    bi = pl.program_id(0)
    ji = pl.program_id(1)
    q = q_ref[0]
    yt = yt_ref[0]
    sq_q = jnp.sum(q * q, axis=1, keepdims=True)
    sq_y = jnp.sum(yt * yt, axis=0, keepdims=True)
    cross = jax.lax.dot_general(
        q, yt, (((1,), (0,)), ((), ())),
        preferred_element_type=jnp.float32)
    d2 = sq_q + sq_y - 2.0 * cross

    inf = jnp.float32(jnp.inf)
    regs4 = _tc_bubble(d2, 4, inf)
    acc, v8 = _tc_extract(list(regs4), inf)
    suspect = jnp.max(jnp.where(regs4[3] <= v8, 1.0, 0.0))

    @pl.when(jnp.logical_and(bi == 0, ji == 0))
    def _():
        out_ref[...] = jnp.zeros_like(out_ref)

    out_ref[...] += acc

    @pl.when(suspect > 0)
    def _():
        acc8, _unused = _tc_extract(_tc_bubble(d2, _K, inf), inf)
        out_ref[...] += acc8 - acc


def kernel(generated):
    generated = generated.astype(jnp.float32)
    b, c, h, w = generated.shape
    n = h * w
    xt = generated.reshape(b * c * n)

    yt_tc = generated.reshape(b, c, n)
    q_tc = jnp.transpose(yt_tc, (0, 2, 1))
    j0 = _RS // _ROWS
    tc_part = pl.pallas_call(
        _tc_body,
        grid=(b, (n - _RS) // _ROWS),
        in_specs=[
            pl.BlockSpec((1, _ROWS, c), lambda i, j: (i, j + j0, 0)),
            pl.BlockSpec((1, c, n), lambda i, j: (i, 0, 0)),
        ],
        out_specs=pl.BlockSpec((_ROWS, 1), lambda i, j: (0, 0)),
        out_shape=jax.ShapeDtypeStruct((_ROWS, 1), jnp.float32),
    )(q_tc, yt_tc)

    mesh = plsc.VectorSubcoreMesh(core_axis_name="c", subcore_axis_name="s")
    run = pl.kernel(
        _sc_body,
        mesh=mesh,
        compiler_params=pltpu.CompilerParams(needs_layout_passes=False),
        out_type=jax.ShapeDtypeStruct((_NW, 16), jnp.float32),
        scratch_types=[
            pltpu.VMEM((n,), jnp.float32),
            pltpu.VMEM((n,), jnp.float32),
            pltpu.VMEM((n,), jnp.float32),
            pltpu.VMEM((n,), jnp.float32),
            pltpu.VMEM((n,), jnp.float32),
            pltpu.VMEM((n,), jnp.float32),
            pltpu.VMEM((n,), jnp.float32),
            pltpu.VMEM((16,), jnp.float32),
        ],
    )
    partial = run(xt)
    total = jnp.sum(partial) + jnp.sum(tc_part)
    return -total / jnp.float32(b * n * _K)


# speedup vs baseline: 1.0879x; 1.0022x over previous
import functools

import jax
import jax.numpy as jnp
from jax import lax
from jax.experimental import pallas as pl
from jax.experimental.pallas import tpu as pltpu
from jax.experimental.pallas import tpu_sc as plsc

_K = 8
_N = 4096
_NW = 32
_RS = 1536
_WPB = 8
_RPW = _RS // _WPB
_NCH = _N // 16
_ROWS = 512
_INF = float("inf")


def _bf16_round(x):
    i = lax.bitcast_convert_type(x, jnp.int32)
    odd = lax.shift_right_logical(i, 16) & jnp.int32(1)
    i = i + jnp.int32(0x7FFF) + odd
    i = i & jnp.int32(-65536)
    return lax.bitcast_convert_type(i, jnp.float32)


def _fsqrt(x):
    i = lax.bitcast_convert_type(x, jnp.int32)
    i = jnp.int32(0x1FBD1DF5) + lax.shift_right_arithmetic(i, 1)
    y = lax.bitcast_convert_type(i, jnp.float32)
    for _ in range(3):
        y = jnp.float32(0.5) * (y + x / y)
    return y


def _sort_asc(v):
    return plsc.sort_key_val(v, v)[0]


def _sort_desc(v):
    return plsc.sort_key_val(v, v, descending=True)[0]


def _merge_sorted(regs):
    b = _sort_asc(regs[0])
    for m in regs[1:]:
        b = _sort_asc(jnp.minimum(b, _sort_desc(m)))
    return b


def _bubble(ms, v):
    out = []
    for m in ms:
        lo = jnp.minimum(m, v)
        v = jnp.maximum(m, v)
        out.append(lo)
    return tuple(out)


def _sc_body(xt_hbm, out_hbm, xs, ys, zs, ss, xr, yr, zr, accs):
    wid = lax.axis_index("s") * 2 + lax.axis_index("c")
    batch = wid // _WPB
    rowbase = (wid % _WPB) * _RPW

    base = batch * 3 * _N
    pltpu.sync_copy(xt_hbm.at[pl.ds(base, _N)], xs)
    pltpu.sync_copy(xt_hbm.at[pl.ds(base + _N, _N)], ys)
    pltpu.sync_copy(xt_hbm.at[pl.ds(base + 2 * _N, _N)], zs)

    lanes = lax.iota(jnp.int32, 16)
    inf16 = jnp.full((16,), _INF, jnp.float32)

    def _norms(j, carry):
        sl = pl.ds(j * 16, 16)
        xv, yv, zv = xs[sl], ys[sl], zs[sl]
        ss[sl] = xv * xv + yv * yv + zv * zv
        xr[sl] = _bf16_round(xv)
        yr[sl] = _bf16_round(yv)
        zr[sl] = _bf16_round(zv)
        return carry

    lax.fori_loop(0, _NCH, _norms, 0)

    def _row(r, acc):
        ridx = jnp.full((16,), rowbase + r, jnp.int32)
        qx = plsc.load_gather(xs, [ridx])
        qy = plsc.load_gather(ys, [ridx])
        qz = plsc.load_gather(zs, [ridx])
        ax = jnp.float32(-2.0) * plsc.load_gather(xr, [ridx])
        ay = jnp.float32(-2.0) * plsc.load_gather(yr, [ridx])
        az = jnp.float32(-2.0) * plsc.load_gather(zr, [ridx])
        sq_q = qx * qx + qy * qy + qz * qz

        def _chunk4(j, ms):
            sl = pl.ds(j * 16, 16)
            v = ss[sl] + ax * xr[sl] + ay * yr[sl] + az * zr[sl]
            return _bubble(ms, v)

        ms4 = lax.fori_loop(0, _NCH, _chunk4, (inf16,) * 4, unroll=8)
        b = _merge_sorted(list(ms4))
        v8 = b[7]
        suspect = plsc.all_reduce_population_count(ms4[3] <= v8)[0]

        def _exact(_):
            def _chunk8(j, ms):
                sl = pl.ds(j * 16, 16)
                v = ss[sl] + ax * xr[sl] + ay * yr[sl] + az * zr[sl]
                return _bubble(ms, v)

            ms8 = lax.fori_loop(0, _NCH, _chunk8, (inf16,) * 8, unroll=4)
            return _merge_sorted(list(ms8))

        vals = lax.cond(suspect > 0, _exact, lambda _: b, 0)

        d2 = jnp.maximum(vals + sq_q, jnp.float32(0.0))
        good = jnp.logical_and(d2 > 0, lanes < _K)
        root = jnp.where(good, _fsqrt(jnp.where(good, d2, 1.0)), 0.0)
        return acc + root

    acc = lax.fori_loop(0, _RPW, _row, jnp.zeros((16,), jnp.float32))
    accs[...] = acc
    pltpu.sync_copy(accs, out_hbm.at[wid])


def _tc_extract(regs, inf, sq_q):
    rows = regs[0].shape[0]
    acc = jnp.zeros((rows, 1), jnp.float32)
    needed = jnp.full((rows, 1), float(_K), jnp.float32)
    m = None
    for _ in range(_K):
        mm = regs[0]
        for reg in regs[1:]:
            mm = jnp.minimum(mm, reg)
        m = jnp.min(mm, axis=1, keepdims=True)
        eqs = [reg == m for reg in regs]
        cnt = jnp.zeros((rows, 1), jnp.float32)
        for eq in eqs:
            cnt = cnt + jnp.sum(eq.astype(jnp.float32), axis=1, keepdims=True)
        take = jnp.minimum(cnt, needed)
        d2m = m + sq_q
        root = jnp.where(d2m > 0, jnp.sqrt(jnp.where(d2m > 0, d2m, 1.0)), 0.0)
        acc = acc + jnp.where(take > 0, take * root, 0.0)
        needed = needed - take
        regs = [jnp.where(eq, inf, reg) for reg, eq in zip(regs, eqs)]
    return acc, m


def _tc_bubble(d2, depth, inf):
    nch = d2.shape[1] // 128
    regs = [jnp.full((d2.shape[0], 128), inf, jnp.float32)
            for _ in range(depth)]
    for c in range(nch):
        v = d2[:, c * 128:(c + 1) * 128]
        out = []
        for m in regs:
            lo = jnp.minimum(m, v)
            v = jnp.maximum(m, v)
            out.append(lo)
        regs = out
    return regs


def _tc_body(q_ref, yt_ref, out_ref):
    bi = pl.program_id(0)
    ji = pl.program_id(1)
    q = q_ref[0]
    yt = yt_ref[0]
    sq_q = jnp.sum(q * q, axis=1, keepdims=True)
    sq_y = jnp.sum(yt * yt, axis=0, keepdims=True)
    cross = jax.lax.dot_general(
        q, yt, (((1,), (0,)), ((), ())),
        preferred_element_type=jnp.float32)
    t = sq_y - 2.0 * cross

    inf = jnp.float32(jnp.inf)
    regs4 = _tc_bubble(t, 4, inf)
    acc, v8 = _tc_extract(list(regs4), inf, sq_q)
    suspect = jnp.max(jnp.where(regs4[3] <= v8, 1.0, 0.0))

    @pl.when(jnp.logical_and(bi == 0, ji == 0))
    def _():
        out_ref[...] = jnp.zeros_like(out_ref)

    out_ref[...] += acc

    @pl.when(suspect > 0)
    def _():
        acc8, _unused = _tc_extract(_tc_bubble(t, _K, inf), inf, sq_q)
        out_ref[...] += acc8 - acc


def kernel(generated):
    generated = generated.astype(jnp.float32)
    b, c, h, w = generated.shape
    n = h * w
    xt = generated.reshape(b * c * n)

    yt_tc = generated.reshape(b, c, n)
    q_tc = jnp.transpose(yt_tc, (0, 2, 1))
    j0 = _RS // _ROWS
    tc_part = pl.pallas_call(
        _tc_body,
        grid=(b, (n - _RS) // _ROWS),
        in_specs=[
            pl.BlockSpec((1, _ROWS, c), lambda i, j: (i, j + j0, 0)),
            pl.BlockSpec((1, c, n), lambda i, j: (i, 0, 0)),
        ],
        out_specs=pl.BlockSpec((_ROWS, 1), lambda i, j: (0, 0)),
        out_shape=jax.ShapeDtypeStruct((_ROWS, 1), jnp.float32),
    )(q_tc, yt_tc)

    mesh = plsc.VectorSubcoreMesh(core_axis_name="c", subcore_axis_name="s")
    run = pl.kernel(
        _sc_body,
        mesh=mesh,
        compiler_params=pltpu.CompilerParams(needs_layout_passes=False),
        out_type=jax.ShapeDtypeStruct((_NW, 16), jnp.float32),
        scratch_types=[
            pltpu.VMEM((n,), jnp.float32),
            pltpu.VMEM((n,), jnp.float32),
            pltpu.VMEM((n,), jnp.float32),
            pltpu.VMEM((n,), jnp.float32),
            pltpu.VMEM((n,), jnp.float32),
            pltpu.VMEM((n,), jnp.float32),
            pltpu.VMEM((n,), jnp.float32),
            pltpu.VMEM((16,), jnp.float32),
        ],
    )
    partial = run(xt)
    total = jnp.sum(partial) + jnp.sum(tc_part)
    return -total / jnp.float32(b * n * _K)


# speedup vs baseline: 1.1303x; 1.0390x over previous
import functools

import jax
import jax.numpy as jnp
from jax import lax
from jax.experimental import pallas as pl
from jax.experimental.pallas import tpu as pltpu
from jax.experimental.pallas import tpu_sc as plsc

_K = 8
_N = 4096
_NW = 32
_RS = 1280
_WPB = 8
_RPW = _RS // _WPB
_NCH = _N // 16
_ROWS = 256
_INF = float("inf")


def _bf16_round(x):
    i = lax.bitcast_convert_type(x, jnp.int32)
    odd = lax.shift_right_logical(i, 16) & jnp.int32(1)
    i = i + jnp.int32(0x7FFF) + odd
    i = i & jnp.int32(-65536)
    return lax.bitcast_convert_type(i, jnp.float32)


def _fsqrt(x):
    i = lax.bitcast_convert_type(x, jnp.int32)
    i = jnp.int32(0x1FBD1DF5) + lax.shift_right_arithmetic(i, 1)
    y = lax.bitcast_convert_type(i, jnp.float32)
    for _ in range(3):
        y = jnp.float32(0.5) * (y + x / y)
    return y


def _sort_asc(v):
    return plsc.sort_key_val(v, v)[0]


def _sort_desc(v):
    return plsc.sort_key_val(v, v, descending=True)[0]


def _merge_sorted(regs):
    b = _sort_asc(regs[0])
    for m in regs[1:]:
        b = _sort_asc(jnp.minimum(b, _sort_desc(m)))
    return b


def _bubble(ms, v):
    out = []
    for m in ms:
        lo = jnp.minimum(m, v)
        v = jnp.maximum(m, v)
        out.append(lo)
    return tuple(out)


def _sc_body(xt_hbm, out_hbm, xs, ys, zs, ss, xr, yr, zr, accs):
    wid = lax.axis_index("s") * 2 + lax.axis_index("c")
    batch = wid // _WPB
    rowbase = (wid % _WPB) * _RPW

    base = batch * 3 * _N
    pltpu.sync_copy(xt_hbm.at[pl.ds(base, _N)], xs)
    pltpu.sync_copy(xt_hbm.at[pl.ds(base + _N, _N)], ys)
    pltpu.sync_copy(xt_hbm.at[pl.ds(base + 2 * _N, _N)], zs)

    lanes = lax.iota(jnp.int32, 16)
    inf16 = jnp.full((16,), _INF, jnp.float32)

    def _norms(j, carry):
        sl = pl.ds(j * 16, 16)
        xv, yv, zv = xs[sl], ys[sl], zs[sl]
        ss[sl] = xv * xv + yv * yv + zv * zv
        xr[sl] = _bf16_round(xv)
        yr[sl] = _bf16_round(yv)
        zr[sl] = _bf16_round(zv)
        return carry

    lax.fori_loop(0, _NCH, _norms, 0)

    def _row(r, acc):
        ridx = jnp.full((16,), rowbase + r, jnp.int32)
        qx = plsc.load_gather(xs, [ridx])
        qy = plsc.load_gather(ys, [ridx])
        qz = plsc.load_gather(zs, [ridx])
        ax = jnp.float32(-2.0) * plsc.load_gather(xr, [ridx])
        ay = jnp.float32(-2.0) * plsc.load_gather(yr, [ridx])
        az = jnp.float32(-2.0) * plsc.load_gather(zr, [ridx])
        sq_q = qx * qx + qy * qy + qz * qz

        def _chunk4(j, ms):
            sl = pl.ds(j * 16, 16)
            v = ss[sl] + ax * xr[sl] + ay * yr[sl] + az * zr[sl]
            return _bubble(ms, v)

        ms4 = lax.fori_loop(0, _NCH, _chunk4, (inf16,) * 4, unroll=8)
        b = _merge_sorted(list(ms4))
        v8 = b[7]
        suspect = plsc.all_reduce_population_count(ms4[3] <= v8)[0]

        def _exact(_):
            def _chunk8(j, ms):
                sl = pl.ds(j * 16, 16)
                v = ss[sl] + ax * xr[sl] + ay * yr[sl] + az * zr[sl]
                return _bubble(ms, v)

            ms8 = lax.fori_loop(0, _NCH, _chunk8, (inf16,) * 8, unroll=4)
            return _merge_sorted(list(ms8))

        vals = lax.cond(suspect > 0, _exact, lambda _: b, 0)

        d2 = jnp.maximum(vals + sq_q, jnp.float32(0.0))
        good = jnp.logical_and(d2 > 0, lanes < _K)
        root = jnp.where(good, _fsqrt(jnp.where(good, d2, 1.0)), 0.0)
        return acc + root

    acc = lax.fori_loop(0, _RPW, _row, jnp.zeros((16,), jnp.float32))
    accs[...] = acc
    pltpu.sync_copy(accs, out_hbm.at[wid])


def _tc_extract(regs, inf, sq_q):
    rows = regs[0].shape[0]
    acc = jnp.zeros((rows, 1), jnp.float32)
    needed = jnp.full((rows, 1), float(_K), jnp.float32)
    m = None
    for _ in range(_K):
        mm = regs[0]
        for reg in regs[1:]:
            mm = jnp.minimum(mm, reg)
        m = jnp.min(mm, axis=1, keepdims=True)
        eqs = [reg == m for reg in regs]
        cnt = jnp.zeros((rows, 1), jnp.float32)
        for eq in eqs:
            cnt = cnt + jnp.sum(eq.astype(jnp.float32), axis=1, keepdims=True)
        take = jnp.minimum(cnt, needed)
        d2m = m + sq_q
        root = jnp.where(d2m > 0, jnp.sqrt(jnp.where(d2m > 0, d2m, 1.0)), 0.0)
        acc = acc + jnp.where(take > 0, take * root, 0.0)
        needed = needed - take
        regs = [jnp.where(eq, inf, reg) for reg, eq in zip(regs, eqs)]
    return acc, m


def _tc_bubble(d2, depth, inf):
    nch = d2.shape[1] // 128
    regs = [jnp.full((d2.shape[0], 128), inf, jnp.float32)
            for _ in range(depth)]
    for c in range(nch):
        v = d2[:, c * 128:(c + 1) * 128]
        out = []
        for m in regs:
            lo = jnp.minimum(m, v)
            v = jnp.maximum(m, v)
            out.append(lo)
        regs = out
    return regs


def _tc_body(q_ref, yt_ref, out_ref):
    bi = pl.program_id(0)
    ji = pl.program_id(1)
    q = q_ref[0]
    yt = yt_ref[0]
    sq_q = jnp.sum(q * q, axis=1, keepdims=True)
    sq_y = jnp.sum(yt * yt, axis=0, keepdims=True)
    cross = jax.lax.dot_general(
        q, yt, (((1,), (0,)), ((), ())),
        preferred_element_type=jnp.float32)
    t = sq_y - 2.0 * cross

    inf = jnp.float32(jnp.inf)
    regs4 = _tc_bubble(t, 4, inf)
    acc, v8 = _tc_extract(list(regs4), inf, sq_q)
    suspect = jnp.max(jnp.where(regs4[3] <= v8, 1.0, 0.0))

    @pl.when(jnp.logical_and(bi == 0, ji == 0))
    def _():
        out_ref[...] = jnp.zeros_like(out_ref)

    out_ref[...] += acc

    @pl.when(suspect > 0)
    def _():
        acc8, _unused = _tc_extract(_tc_bubble(t, _K, inf), inf, sq_q)
        out_ref[...] += acc8 - acc


def kernel(generated):
    generated = generated.astype(jnp.float32)
    b, c, h, w = generated.shape
    n = h * w
    xt = generated.reshape(b * c * n)

    yt_tc = generated.reshape(b, c, n)
    q_tc = jnp.transpose(yt_tc, (0, 2, 1))
    j0 = _RS // _ROWS
    tc_part = pl.pallas_call(
        _tc_body,
        grid=(b, (n - _RS) // _ROWS),
        in_specs=[
            pl.BlockSpec((1, _ROWS, c), lambda i, j: (i, j + j0, 0)),
            pl.BlockSpec((1, c, n), lambda i, j: (i, 0, 0)),
        ],
        out_specs=pl.BlockSpec((_ROWS, 1), lambda i, j: (0, 0)),
        out_shape=jax.ShapeDtypeStruct((_ROWS, 1), jnp.float32),
    )(q_tc, yt_tc)

    mesh = plsc.VectorSubcoreMesh(core_axis_name="c", subcore_axis_name="s")
    run = pl.kernel(
        _sc_body,
        mesh=mesh,
        compiler_params=pltpu.CompilerParams(needs_layout_passes=False),
        out_type=jax.ShapeDtypeStruct((_NW, 16), jnp.float32),
        scratch_types=[
            pltpu.VMEM((n,), jnp.float32),
            pltpu.VMEM((n,), jnp.float32),
            pltpu.VMEM((n,), jnp.float32),
            pltpu.VMEM((n,), jnp.float32),
            pltpu.VMEM((n,), jnp.float32),
            pltpu.VMEM((n,), jnp.float32),
            pltpu.VMEM((n,), jnp.float32),
            pltpu.VMEM((16,), jnp.float32),
        ],
    )
    partial = run(xt)
    total = jnp.sum(partial) + jnp.sum(tc_part)
    return -total / jnp.float32(b * n * _K)
